# baseline (speedup 1.0000x reference)
import functools

import jax
import jax.numpy as jnp
from jax import lax
from jax.experimental import pallas as pl
from jax.experimental.pallas import tpu as pltpu
from jax.experimental.pallas import tpu_sc as plsc

EPS = 1e-12
L = 16


def _rsqrt(x):
    xi = lax.bitcast_convert_type(x, jnp.int32)
    yi = jnp.int32(0x5F3759DF) - lax.shift_right_arithmetic(xi, 1)
    y = lax.bitcast_convert_type(yi, jnp.float32)
    for _ in range(3):
        y = y * (1.5 - 0.5 * x * y * y)
    return y


def kernel(input_tag_ids, table, ln_weight, ln_bias):
    B, S = input_tag_ids.shape
    V, D = table.shape
    N = B * S
    NC, NS = 2, 16
    NW = NC * NS
    CHUNK = 128
    per_w = N // NW
    n_chunks = per_w // CHUNK
    assert per_w * NW == N and n_chunks * CHUNK == per_w and D == 2 * L

    idx_flat = input_tag_ids.reshape(NW, n_chunks, CHUNK)
    mesh = plsc.VectorSubcoreMesh(core_axis_name="c", subcore_axis_name="s")

    @functools.partial(
        pl.kernel,
        mesh=mesh,
        compiler_params=pltpu.CompilerParams(
            needs_layout_passes=False, use_tc_tiling_on_sc=False
        ),
        out_type=jax.ShapeDtypeStruct((N, D), jnp.float32),
        scratch_types=[
            pltpu.VMEM((n_chunks, CHUNK), jnp.int32),
            pltpu.VMEM((CHUNK, D), jnp.float32),
            pltpu.VMEM((D,), jnp.float32),
            pltpu.VMEM((D,), jnp.float32),
            pltpu.SemaphoreType.DMA,
        ],
    )
    def k(idx_hbm, table_hbm, w_hbm, b_hbm, out_hbm, idx_v, rows_v, w_v, b_v, sem):
        wid = lax.axis_index("s") * NC + lax.axis_index("c")
        pltpu.sync_copy(idx_hbm.at[wid], idx_v)
        pltpu.sync_copy(w_hbm, w_v)
        pltpu.sync_copy(b_hbm, b_v)
        w_lo, w_hi = w_v[pl.ds(0, L)], w_v[pl.ds(L, L)]
        b_lo, b_hi = b_v[pl.ds(0, L)], b_v[pl.ds(L, L)]
        w_sc = [w_lo[c] for c in range(L)] + [w_hi[c] for c in range(L)]
        b_sc = [b_lo[c] for c in range(L)] + [b_hi[c] for c in range(L)]

        def chunk_body(j, _):
            pltpu.async_copy(table_hbm.at[idx_v.at[j]], rows_v, sem).wait()

            def group_body(g, _):
                rows16 = g * L + lax.iota(jnp.int32, L)
                cols = []
                s1 = jnp.zeros((L,), jnp.float32)
                for c in range(D):
                    v = plsc.load_gather(
                        rows_v, [rows16, jnp.full((L,), c, jnp.int32)]
                    )
                    cols.append(v)
                    s1 = s1 + v
                u = s1 * (1.0 / D)
                s2 = jnp.zeros((L,), jnp.float32)
                for c in range(D):
                    cols[c] = cols[c] - u
                    s2 = s2 + cols[c] * cols[c]
                inv = _rsqrt(jnp.maximum(s2 * (1.0 / D), 0.0) + EPS)
                for c in range(D):
                    plsc.store_scatter(
                        rows_v,
                        [rows16, jnp.full((L,), c, jnp.int32)],
                        cols[c] * inv * w_sc[c] + b_sc[c],
                    )
                return 0

            lax.fori_loop(0, CHUNK // L, group_body, 0)
            pltpu.sync_copy(rows_v, out_hbm.at[pl.ds((wid * n_chunks + j) * CHUNK, CHUNK)])
            return 0

        lax.fori_loop(0, n_chunks, chunk_body, 0)

    out = k(idx_flat, table, ln_weight, ln_bias)
    return out.reshape(B, S, D)


# speedup vs baseline: 1.1850x; 1.1850x over previous
import functools

import jax
import jax.numpy as jnp
from jax import lax
from jax.experimental import pallas as pl
from jax.experimental.pallas import tpu as pltpu
from jax.experimental.pallas import tpu_sc as plsc

EPS = 1e-12
L = 16
GATHER = 128
NBUF = 2


def _rsqrt(x):
    xi = lax.bitcast_convert_type(x, jnp.int32)
    yi = jnp.int32(0x5F3759DF) - lax.shift_right_arithmetic(xi, 1)
    y = lax.bitcast_convert_type(yi, jnp.float32)
    for _ in range(3):
        y = y * (1.5 - 0.5 * x * y * y)
    return y


def kernel(input_tag_ids, table, ln_weight, ln_bias):
    B, S = input_tag_ids.shape
    V, D = table.shape
    N = B * S
    NC, NS = 2, 16
    NW = NC * NS
    per_w = N // NW
    SUPER = 512
    GPS = SUPER // GATHER
    n_super = per_w // SUPER
    n_gather = per_w // GATHER
    assert per_w * NW == N and n_super * SUPER == per_w and D == 2 * L
    assert n_super >= 2 * NBUF

    idx_flat = input_tag_ids.reshape(NW, n_gather, GATHER)
    mesh = plsc.VectorSubcoreMesh(core_axis_name="c", subcore_axis_name="s")

    @functools.partial(
        pl.kernel,
        mesh=mesh,
        compiler_params=pltpu.CompilerParams(
            needs_layout_passes=False, use_tc_tiling_on_sc=False
        ),
        out_type=jax.ShapeDtypeStruct((N, D), jnp.float32),
        scratch_types=[
            pltpu.VMEM((n_gather, GATHER), jnp.int32),
            pltpu.VMEM((SUPER, D), jnp.float32),
            pltpu.VMEM((SUPER, D), jnp.float32),
            pltpu.VMEM((SUPER, D), jnp.float32),
            pltpu.VMEM((SUPER, D), jnp.float32),
            pltpu.VMEM((D,), jnp.float32),
            pltpu.VMEM((D,), jnp.float32),
            pltpu.SemaphoreType.DMA,
            pltpu.SemaphoreType.DMA,
            pltpu.SemaphoreType.DMA,
            pltpu.SemaphoreType.DMA,
        ],
    )
    def k(idx_hbm, table_hbm, w_hbm, b_hbm, out_hbm,
          idx_v, rows0, rows1, obuf0, obuf1, w_v, b_v, g0, g1, o0, o1):
        wid = lax.axis_index("s") * NC + lax.axis_index("c")
        rows = [rows0, rows1]
        obuf = [obuf0, obuf1]
        gsem = [g0, g1]
        osem = [o0, o1]
        pltpu.sync_copy(idx_hbm.at[wid], idx_v)
        pltpu.sync_copy(w_hbm, w_v)
        pltpu.sync_copy(b_hbm, b_v)
        w_lo, w_hi = w_v[pl.ds(0, L)], w_v[pl.ds(L, L)]
        b_lo, b_hi = b_v[pl.ds(0, L)], b_v[pl.ds(L, L)]
        w_sc = [w_lo[c] for c in range(L)] + [w_hi[c] for c in range(L)]
        b_sc = [b_lo[c] for c in range(L)] + [b_hi[c] for c in range(L)]

        def issue_gather(j, b):
            for i in range(GPS):
                pltpu.async_copy(
                    table_hbm.at[idx_v.at[GPS * j + i]],
                    rows[b].at[pl.ds(i * GATHER, GATHER)],
                    gsem[b],
                )

        def wait_gather(j, b):
            for i in range(GPS):
                pltpu.make_async_copy(
                    table_hbm.at[idx_v.at[GPS * j + i]],
                    rows[b].at[pl.ds(i * GATHER, GATHER)],
                    gsem[b],
                ).wait()

        def out_slice(j):
            return out_hbm.at[pl.ds((wid * n_super + j) * SUPER, SUPER)]

        def issue_out(j, b):
            pltpu.async_copy(obuf[b], out_slice(j), osem[b])

        def wait_out(j, b):
            pltpu.make_async_copy(obuf[b], out_slice(j), osem[b]).wait()

        def compute(b):
            src, dst = rows[b], obuf[b]

            def group_body(g, _):
                rows16 = g * L + lax.iota(jnp.int32, L)
                cols = []
                part = [jnp.zeros((L,), jnp.float32) for _ in range(4)]
                for c in range(D):
                    v = plsc.load_gather(
                        src, [rows16, jnp.full((L,), c, jnp.int32)]
                    )
                    cols.append(v)
                    part[c % 4] = part[c % 4] + v
                u = ((part[0] + part[1]) + (part[2] + part[3])) * (1.0 / D)
                part = [jnp.zeros((L,), jnp.float32) for _ in range(4)]
                for c in range(D):
                    cols[c] = cols[c] - u
                    part[c % 4] = part[c % 4] + cols[c] * cols[c]
                s2 = (part[0] + part[1]) + (part[2] + part[3])
                inv = _rsqrt(jnp.maximum(s2 * (1.0 / D), 0.0) + EPS)
                for c in range(D):
                    plsc.store_scatter(
                        dst,
                        [rows16, jnp.full((L,), c, jnp.int32)],
                        cols[c] * inv * w_sc[c] + b_sc[c],
                    )
                return 0

            lax.fori_loop(0, SUPER // L, group_body, 0)

        issue_gather(0, 0)
        issue_gather(1, 1)

        for j in range(NBUF):
            b = j % NBUF
            wait_gather(j, b)
            compute(b)
            issue_out(j, b)
            issue_gather(j + NBUF, b)

        def steady(i, _):
            j0 = NBUF + i * NBUF
            for b in range(NBUF):
                j = j0 + b
                wait_gather(j, b)
                wait_out(j - NBUF, b)
                compute(b)
                issue_out(j, b)
                issue_gather(j + NBUF, b)
            return 0

        lax.fori_loop(0, (n_super - 2 * NBUF) // NBUF, steady, 0)

        for j in range(n_super - NBUF, n_super):
            b = j % NBUF
            wait_gather(j, b)
            wait_out(j - NBUF, b)
            compute(b)
            issue_out(j, b)
        for j in range(n_super - NBUF, n_super):
            wait_out(j, j % NBUF)

    out = k(idx_flat, table, ln_weight, ln_bias)
    return out.reshape(B, S, D)


# speedup vs baseline: 1.8564x; 1.5666x over previous
import functools

import jax
import jax.numpy as jnp
from jax import lax
from jax.experimental import pallas as pl
from jax.experimental.pallas import tpu as pltpu
from jax.experimental.pallas import tpu_sc as plsc

EPS = 1e-12
L = 16
GATHER = 128
NBUF = 2
BCHUNK = 512


def _rsqrt(x):
    xi = lax.bitcast_convert_type(x, jnp.int32)
    yi = jnp.int32(0x5F3759DF) - lax.shift_right_arithmetic(xi, 1)
    y = lax.bitcast_convert_type(yi, jnp.float32)
    for _ in range(3):
        y = y * (1.5 - 0.5 * x * y * y)
    return y


def kernel(input_tag_ids, table, ln_weight, ln_bias):
    B, S = input_tag_ids.shape
    V, D = table.shape
    NC, NS = 2, 16
    NW = NC * NS
    n_bc = B // BCHUNK
    n_sg = NW // n_bc
    n_super = S // n_sg
    GPS = BCHUNK // GATHER
    assert n_bc * BCHUNK == B and n_sg * n_super == S and D == 2 * L
    assert n_super >= 2 * NBUF

    ids_t = input_tag_ids.T
    mesh = plsc.VectorSubcoreMesh(core_axis_name="c", subcore_axis_name="s")

    @functools.partial(
        pl.kernel,
        mesh=mesh,
        compiler_params=pltpu.CompilerParams(
            needs_layout_passes=False, use_tc_tiling_on_sc=False
        ),
        out_type=jax.ShapeDtypeStruct((S, D, B), jnp.float32),
        scratch_types=[
            pltpu.VMEM((n_super, BCHUNK), jnp.int32),
            pltpu.VMEM((BCHUNK, D), jnp.float32),
            pltpu.VMEM((BCHUNK, D), jnp.float32),
            pltpu.VMEM((D, BCHUNK), jnp.float32),
            pltpu.VMEM((D, BCHUNK), jnp.float32),
            pltpu.VMEM((D,), jnp.float32),
            pltpu.VMEM((D,), jnp.float32),
            pltpu.SemaphoreType.DMA,
            pltpu.SemaphoreType.DMA,
            pltpu.SemaphoreType.DMA,
            pltpu.SemaphoreType.DMA,
            pltpu.SemaphoreType.DMA,
        ],
    )
    def k(idx_hbm, table_hbm, w_hbm, b_hbm, out_hbm,
          idx_v, rows0, rows1, obuf0, obuf1, w_v, b_v, g0, g1, o0, o1, isem):
        wid = lax.axis_index("s") * NC + lax.axis_index("c")
        sg = wid // n_bc
        bc = wid % n_bc
        b0 = bc * BCHUNK
        rows = [rows0, rows1]
        obuf = [obuf0, obuf1]
        gsem = [g0, g1]
        osem = [o0, o1]

        def idx_issue(t, _):
            pltpu.async_copy(
                idx_hbm.at[sg * n_super + t, pl.ds(b0, BCHUNK)],
                idx_v.at[t], isem)
            return 0

        def idx_drain(t, _):
            pltpu.make_async_copy(
                idx_hbm.at[sg * n_super + t, pl.ds(b0, BCHUNK)],
                idx_v.at[t], isem).wait()
            return 0

        lax.fori_loop(0, n_super, idx_issue, 0)
        pltpu.sync_copy(w_hbm, w_v)
        pltpu.sync_copy(b_hbm, b_v)
        lax.fori_loop(0, n_super, idx_drain, 0)
        w_lo, w_hi = w_v[pl.ds(0, L)], w_v[pl.ds(L, L)]
        b_lo, b_hi = b_v[pl.ds(0, L)], b_v[pl.ds(L, L)]
        w_sc = [w_lo[c] for c in range(L)] + [w_hi[c] for c in range(L)]
        b_sc = [b_lo[c] for c in range(L)] + [b_hi[c] for c in range(L)]

        def issue_gather(j, b):
            for i in range(GPS):
                pltpu.async_copy(
                    table_hbm.at[idx_v.at[j, pl.ds(i * GATHER, GATHER)]],
                    rows[b].at[pl.ds(i * GATHER, GATHER)],
                    gsem[b],
                )

        def wait_gather(j, b):
            for i in range(GPS):
                pltpu.make_async_copy(
                    table_hbm.at[idx_v.at[j, pl.ds(i * GATHER, GATHER)]],
                    rows[b].at[pl.ds(i * GATHER, GATHER)],
                    gsem[b],
                ).wait()

        def out_slice(j):
            return out_hbm.at[sg * n_super + j, :, pl.ds(b0, BCHUNK)]

        def issue_out(j, b):
            pltpu.async_copy(obuf[b], out_slice(j), osem[b])

        def wait_out(j, b):
            pltpu.make_async_copy(obuf[b], out_slice(j), osem[b]).wait()

        def compute(b):
            src, dst = rows[b], obuf[b]

            def group_body(g, _):
                t0 = g * L
                rows16 = t0 + lax.iota(jnp.int32, L)
                cols = []
                part = [jnp.zeros((L,), jnp.float32) for _ in range(4)]
                for c in range(D):
                    v = plsc.load_gather(
                        src, [rows16, jnp.full((L,), c, jnp.int32)]
                    )
                    cols.append(v)
                    part[c % 4] = part[c % 4] + v
                u = ((part[0] + part[1]) + (part[2] + part[3])) * (1.0 / D)
                part = [jnp.zeros((L,), jnp.float32) for _ in range(4)]
                for c in range(D):
                    cols[c] = cols[c] - u
                    part[c % 4] = part[c % 4] + cols[c] * cols[c]
                s2 = (part[0] + part[1]) + (part[2] + part[3])
                inv = _rsqrt(jnp.maximum(s2 * (1.0 / D), 0.0) + EPS)
                for c in range(D):
                    dst[c, pl.ds(t0, L)] = cols[c] * inv * w_sc[c] + b_sc[c]
                return 0

            lax.fori_loop(0, BCHUNK // L, group_body, 0)

        issue_gather(0, 0)
        issue_gather(1, 1)

        for j in range(NBUF):
            b = j % NBUF
            wait_gather(j, b)
            compute(b)
            issue_out(j, b)
            issue_gather(j + NBUF, b)

        def steady(i, _):
            j0 = NBUF + i * NBUF
            for b in range(NBUF):
                j = j0 + b
                wait_gather(j, b)
                wait_out(j - NBUF, b)
                compute(b)
                issue_out(j, b)
                issue_gather(j + NBUF, b)
            return 0

        lax.fori_loop(0, (n_super - 2 * NBUF) // NBUF, steady, 0)

        for j in range(n_super - NBUF, n_super):
            b = j % NBUF
            wait_gather(j, b)
            wait_out(j - NBUF, b)
            compute(b)
            issue_out(j, b)
        for j in range(n_super - NBUF, n_super):
            wait_out(j, j % NBUF)

    out = k(ids_t, table, ln_weight, ln_bias)
    return jnp.transpose(out, (2, 0, 1))


# speedup vs baseline: 2.0280x; 1.0925x over previous
import functools

import jax
import jax.numpy as jnp
from jax import lax
from jax.experimental import pallas as pl
from jax.experimental.pallas import tpu as pltpu
from jax.experimental.pallas import tpu_sc as plsc

EPS = 1e-12
L = 16
BCHUNK = 128
NBUF = 2
BLK = 4


def _rsqrt(x):
    xi = lax.bitcast_convert_type(x, jnp.int32)
    yi = jnp.int32(0x5F3759DF) - lax.shift_right_arithmetic(xi, 1)
    y = lax.bitcast_convert_type(yi, jnp.float32)
    for _ in range(3):
        y = y * (1.5 - 0.5 * x * y * y)
    return y


def kernel(input_tag_ids, table, ln_weight, ln_bias):
    B, S = input_tag_ids.shape
    V, D = table.shape
    NC, NS = 2, 16
    NW = NC * NS
    n_super = S
    assert NW * BCHUNK == B and D == 2 * L and BLK * D == 128

    ids_t = input_tag_ids.T
    tbl_blk = table.reshape(V // BLK, BLK * D)
    mesh = plsc.VectorSubcoreMesh(core_axis_name="c", subcore_axis_name="s")

    @functools.partial(
        pl.kernel,
        mesh=mesh,
        compiler_params=pltpu.CompilerParams(
            needs_layout_passes=False, use_tc_tiling_on_sc=True
        ),
        out_type=jax.ShapeDtypeStruct((S, D, B), jnp.float32),
        scratch_types=[
            pltpu.VMEM((n_super, BCHUNK), jnp.int32),
            pltpu.VMEM((NBUF, BCHUNK), jnp.int32),
            pltpu.VMEM((BCHUNK, BLK * D), jnp.float32),
            pltpu.VMEM((BCHUNK, BLK * D), jnp.float32),
            pltpu.VMEM((D, BCHUNK), jnp.float32),
            pltpu.VMEM((D, BCHUNK), jnp.float32),
            pltpu.VMEM((D,), jnp.float32),
            pltpu.VMEM((D,), jnp.float32),
            pltpu.SemaphoreType.DMA,
            pltpu.SemaphoreType.DMA,
            pltpu.SemaphoreType.DMA,
            pltpu.SemaphoreType.DMA,
            pltpu.SemaphoreType.DMA,
        ],
    )
    def k(idx_hbm, table_hbm, w_hbm, b_hbm, out_hbm,
          idx_v, blk_v, rows0, rows1, obuf0, obuf1, w_v, b_v,
          g0, g1, o0, o1, isem):
        wid = lax.axis_index("s") * NC + lax.axis_index("c")
        b0 = wid * BCHUNK
        rows = [rows0, rows1]
        obuf = [obuf0, obuf1]
        gsem = [g0, g1]
        osem = [o0, o1]

        def idx_issue(t, _):
            pltpu.async_copy(
                idx_hbm.at[t, pl.ds(b0, BCHUNK)], idx_v.at[t], isem)
            return 0

        def idx_drain(t, _):
            pltpu.make_async_copy(
                idx_hbm.at[t, pl.ds(b0, BCHUNK)], idx_v.at[t], isem).wait()
            return 0

        lax.fori_loop(0, n_super, idx_issue, 0)
        pltpu.sync_copy(w_hbm, w_v)
        pltpu.sync_copy(b_hbm, b_v)
        lax.fori_loop(0, n_super, idx_drain, 0)
        w_lo, w_hi = w_v[pl.ds(0, L)], w_v[pl.ds(L, L)]
        b_lo, b_hi = b_v[pl.ds(0, L)], b_v[pl.ds(L, L)]
        w_sc = [w_lo[c] for c in range(L)] + [w_hi[c] for c in range(L)]
        b_sc = [b_lo[c] for c in range(L)] + [b_hi[c] for c in range(L)]

        def issue_gather(j, b):
            for q in range(BCHUNK // L):
                blk_v[b, pl.ds(q * L, L)] = lax.shift_right_logical(
                    idx_v[j, pl.ds(q * L, L)], 2)
            pltpu.async_copy(
                table_hbm.at[blk_v.at[b]], rows[b], gsem[b])

        def wait_gather(b):
            pltpu.make_async_copy(
                table_hbm.at[blk_v.at[b]], rows[b], gsem[b]).wait()

        def out_slice(j):
            return out_hbm.at[j, :, pl.ds(b0, BCHUNK)]

        def issue_out(j, b):
            pltpu.async_copy(obuf[b], out_slice(j), osem[b])

        def wait_out(j, b):
            pltpu.make_async_copy(obuf[b], out_slice(j), osem[b]).wait()

        def compute(j, b):
            src, dst = rows[b], obuf[b]

            def group_body(g, _):
                t0 = g * L
                rows16 = t0 + lax.iota(jnp.int32, L)
                sub = (idx_v[j, pl.ds(t0, L)] & 3) * D
                cols = []
                part = [jnp.zeros((L,), jnp.float32) for _ in range(4)]
                for c in range(D):
                    v = plsc.load_gather(src, [rows16, sub + c])
                    cols.append(v)
                    part[c % 4] = part[c % 4] + v
                u = ((part[0] + part[1]) + (part[2] + part[3])) * (1.0 / D)
                part = [jnp.zeros((L,), jnp.float32) for _ in range(4)]
                for c in range(D):
                    cols[c] = cols[c] - u
                    part[c % 4] = part[c % 4] + cols[c] * cols[c]
                s2 = (part[0] + part[1]) + (part[2] + part[3])
                inv = _rsqrt(jnp.maximum(s2 * (1.0 / D), 0.0) + EPS)
                for c in range(D):
                    dst[c, pl.ds(t0, L)] = cols[c] * inv * w_sc[c] + b_sc[c]
                return 0

            lax.fori_loop(0, BCHUNK // L, group_body, 0)

        issue_gather(0, 0)
        issue_gather(1, 1)

        for j in range(NBUF):
            b = j % NBUF
            wait_gather(b)
            compute(j, b)
            issue_out(j, b)
            issue_gather(j + NBUF, b)

        def steady(i, _):
            j0 = NBUF + i * NBUF
            for b in range(NBUF):
                j = j0 + b
                wait_gather(b)
                wait_out(j - NBUF, b)
                compute(j, b)
                issue_out(j, b)
                issue_gather(j + NBUF, b)
            return 0

        lax.fori_loop(0, (n_super - 2 * NBUF) // NBUF, steady, 0)

        for j in range(n_super - NBUF, n_super):
            b = j % NBUF
            wait_gather(b)
            wait_out(j - NBUF, b)
            compute(j, b)
            issue_out(j, b)
        for j in range(n_super - NBUF, n_super):
            wait_out(j, j % NBUF)

    out = k(ids_t, tbl_blk, ln_weight, ln_bias)
    return jnp.transpose(out, (2, 0, 1))
